# Initial kernel scaffold; baseline (speedup 1.0000x reference)
import jax
import jax.numpy as jnp
from jax.experimental import pallas as pl


def kernel(pred_logits, pred_boxes, tgt_boxes, tgt_labels, src_idx, tgt_idx):
    raise NotImplementedError("write your pallas kernel here")


# speedup vs baseline: 1.2210x; 1.2210x over previous
import functools
import math

import jax
import jax.numpy as jnp
from jax import lax
from jax.experimental import pallas as pl
from jax.experimental.pallas import tpu as pltpu
from jax.experimental.pallas import tpu_sc as plsc

_B, _N, _M = 16, 500, 64
_NPAD = 512
_L = 16
_LN2 = math.log(2.0)


def _sc_body(x_hbm, pb_hbm, tb_hbm, lab_hbm, src_hbm, tgt_hbm,
             part_hbm, out_hbm,
             xv, zv, pbv, tbv, labv, srcv, tgtv, partv, allv, outv):
    cid = lax.axis_index("c")
    sid = lax.axis_index("s")
    row = sid

    pltpu.sync_copy(x_hbm.at[row], xv)
    pltpu.sync_copy(pb_hbm.at[row], pbv)
    pltpu.sync_copy(tb_hbm.at[row], tbv)
    pltpu.sync_copy(lab_hbm.at[row], labv)
    pltpu.sync_copy(src_hbm.at[row], srcv)
    pltpu.sync_copy(tgt_hbm.at[row], tgtv)

    zero = jnp.zeros((_L,), jnp.float32)
    for k in range(_NPAD // _L):
        zv[pl.ds(k * _L, _L)] = zero

    for k in range(_M // _L):
        svec = srcv[pl.ds(k * _L, _L)]
        tvec = tgtv[pl.ds(k * _L, _L)]
        lab = plsc.load_gather(labv, [tvec]).astype(jnp.float32)
        plsc.store_scatter(zv, [svec], lab)

    acc_ce = zero
    for k in range(_NPAD // _L):
        x = xv[pl.ds(k * _L, _L)]
        z = zv[pl.ds(k * _L, _L)]
        u = jnp.exp(-jnp.abs(x))
        s = u / (u + 2.0)
        s2 = s * s
        lg = s * (2.0 + s2 * (2.0 / 3.0 + s2 * (2.0 / 5.0 + s2 * (2.0 / 7.0))))
        acc_ce = acc_ce + (jnp.maximum(x, 0.0) - x * z + lg)

    acc_bb = zero
    for k in range(_M // _L):
        svec = srcv[pl.ds(k * _L, _L)]
        tvec = tgtv[pl.ds(k * _L, _L)]
        for c in range(3):
            cvec = jnp.full((_L,), c, jnp.int32)
            sp = plsc.load_gather(pbv, [svec, cvec])
            tp = plsc.load_gather(tbv, [tvec, cvec])
            acc_bb = acc_bb + jnp.abs(sp - tp)

    partv[0, :] = acc_ce
    partv[1, :] = acc_bb
    pltpu.sync_copy(partv, part_hbm.at[sid])
    plsc.subcore_barrier()

    @pl.when(jnp.logical_and(cid == 0, sid == 0))
    def _():
        pltpu.sync_copy(part_hbm, allv)
        cev = jnp.zeros((_L,), jnp.float32)
        bbv = jnp.zeros((_L,), jnp.float32)
        for s_ in range(_B):
            cev = cev + allv[s_, 0, :]
            bbv = bbv + allv[s_, 1, :]
        ce_sum = jnp.sum(cev)
        bb_sum = jnp.sum(bbv)
        loss_ce = (ce_sum - _B * (_NPAD - _N) * _LN2) * (1.0 / (_B * _N))
        loss_bb = bb_sum * (1.0 / (_B * _M))
        lane = lax.iota(jnp.int32, _L)
        res = jnp.where(lane == 0, loss_ce, jnp.where(lane == 1, loss_bb, 0.0))
        outv[...] = res
        pltpu.sync_copy(outv, out_hbm)


_SCRATCH = [
    pltpu.VMEM((_NPAD,), jnp.float32),
    pltpu.VMEM((_NPAD,), jnp.float32),
    pltpu.VMEM((_NPAD, 3), jnp.float32),
    pltpu.VMEM((_M, 3), jnp.float32),
    pltpu.VMEM((_M,), jnp.int32),
    pltpu.VMEM((_M,), jnp.int32),
    pltpu.VMEM((_M,), jnp.int32),
    pltpu.VMEM((2, _L), jnp.float32),
    pltpu.VMEM((_B, 2, _L), jnp.float32),
    pltpu.VMEM((_L,), jnp.float32),
]


def _make_kernel(interpret=False):
    return pl.kernel(
        _sc_body,
        out_type=(jax.ShapeDtypeStruct((_B, 2, _L), jnp.float32),
                  jax.ShapeDtypeStruct((_L,), jnp.float32)),
        mesh=plsc.VectorSubcoreMesh(
            core_axis_name="c", subcore_axis_name="s",
            num_cores=2, num_subcores=16),
        scratch_types=_SCRATCH,
        compiler_params=pltpu.CompilerParams(needs_layout_passes=False),
        interpret=interpret,
    )


def kernel(pred_logits, pred_boxes, tgt_boxes, tgt_labels, src_idx, tgt_idx):
    x = jnp.squeeze(pred_logits, axis=-1)
    xp = jnp.pad(x, ((0, 0), (0, _NPAD - _N)))
    pbp = jnp.pad(pred_boxes, ((0, 0), (0, _NPAD - _N), (0, 0)))
    _, out = _make_kernel()(
        xp.astype(jnp.float32),
        pbp.astype(jnp.float32),
        tgt_boxes.astype(jnp.float32),
        tgt_labels.astype(jnp.int32),
        src_idx.astype(jnp.int32),
        tgt_idx.astype(jnp.int32),
    )
    return (out[0], out[1])


# speedup vs baseline: 1.3012x; 1.0657x over previous
import functools
import math

import jax
import jax.numpy as jnp
from jax import lax
from jax.experimental import pallas as pl
from jax.experimental.pallas import tpu as pltpu
from jax.experimental.pallas import tpu_sc as plsc

_B, _N, _M = 16, 500, 64
_L = 16
_NCHUNK = _N // _L
_TAIL = _N - _NCHUNK * _L
_TBASE = _N - _L


def _sc_body(x_hbm, pb_hbm, tb_hbm, lab_hbm, src_hbm, tgt_hbm,
             part_hbm, out_hbm,
             xv, zv, pbv, tbv, labv, srcv, tgtv, partv, allv, outv, sem):
    cid = lax.axis_index("c")
    sid = lax.axis_index("s")
    row = sid

    cps = [
        pltpu.async_copy(x_hbm.at[row], xv, sem),
        pltpu.async_copy(pb_hbm.at[row], pbv, sem),
        pltpu.async_copy(tb_hbm.at[row], tbv, sem),
        pltpu.async_copy(lab_hbm.at[row], labv, sem),
        pltpu.async_copy(src_hbm.at[row], srcv, sem),
        pltpu.async_copy(tgt_hbm.at[row], tgtv, sem),
    ]

    zero = jnp.zeros((_L,), jnp.float32)
    for k in range(_N // _L + 1):
        zv[pl.ds(k * _L, _L)] = zero

    for cp in cps:
        cp.wait()

    for k in range(_M // _L):
        svec = srcv[pl.ds(k * _L, _L)]
        tvec = tgtv[pl.ds(k * _L, _L)]
        lab = plsc.load_gather(labv, [tvec]).astype(jnp.float32)
        plsc.store_scatter(zv, [svec], lab)

    def bce(x, z):
        u = jnp.exp(-jnp.abs(x))
        s = u / (u + 2.0)
        s2 = s * s
        lg = s * (2.0 + s2 * (2.0 / 3.0 + s2 * (2.0 / 5.0 + s2 * (2.0 / 7.0))))
        return jnp.maximum(x, 0.0) - x * z + lg

    acc_ce = zero
    for k in range(_NCHUNK):
        acc_ce = acc_ce + bce(xv[pl.ds(k * _L, _L)], zv[pl.ds(k * _L, _L)])
    lane = lax.iota(jnp.int32, _L)
    tail = bce(xv[pl.ds(_TBASE, _L)], zv[pl.ds(_TBASE, _L)])
    acc_ce = acc_ce + jnp.where(lane >= _L - _TAIL, tail, 0.0)

    acc_bb = zero
    for k in range(_M // _L):
        svec = srcv[pl.ds(k * _L, _L)]
        tvec = tgtv[pl.ds(k * _L, _L)]
        for c in range(3):
            cvec = jnp.full((_L,), c, jnp.int32)
            sp = plsc.load_gather(pbv, [svec, cvec])
            tp = plsc.load_gather(tbv, [tvec, cvec])
            acc_bb = acc_bb + jnp.abs(sp - tp)

    partv[0, :] = acc_ce
    partv[1, :] = acc_bb
    pltpu.sync_copy(partv, part_hbm.at[sid])
    plsc.subcore_barrier()

    @pl.when(jnp.logical_and(cid == 0, sid == 0))
    def _():
        pltpu.sync_copy(part_hbm, allv)
        cev = jnp.zeros((_L,), jnp.float32)
        bbv = jnp.zeros((_L,), jnp.float32)
        for s_ in range(_B):
            cev = cev + allv[s_, 0, :]
            bbv = bbv + allv[s_, 1, :]
        ce_sum = jnp.sum(cev)
        bb_sum = jnp.sum(bbv)
        loss_ce = ce_sum * (1.0 / (_B * _N))
        loss_bb = bb_sum * (1.0 / (_B * _M))
        res = jnp.where(lane == 0, loss_ce, jnp.where(lane == 1, loss_bb, 0.0))
        outv[...] = res
        pltpu.sync_copy(outv, out_hbm)


_SCRATCH = [
    pltpu.VMEM((_N,), jnp.float32),
    pltpu.VMEM((_N + _L, ), jnp.float32),
    pltpu.VMEM((_N, 3), jnp.float32),
    pltpu.VMEM((_M, 3), jnp.float32),
    pltpu.VMEM((_M,), jnp.int32),
    pltpu.VMEM((_M,), jnp.int32),
    pltpu.VMEM((_M,), jnp.int32),
    pltpu.VMEM((2, _L), jnp.float32),
    pltpu.VMEM((_B, 2, _L), jnp.float32),
    pltpu.VMEM((_L,), jnp.float32),
    pltpu.SemaphoreType.DMA,
]


def _make_kernel(interpret=False):
    return pl.kernel(
        _sc_body,
        out_type=(jax.ShapeDtypeStruct((_B, 2, _L), jnp.float32),
                  jax.ShapeDtypeStruct((_L,), jnp.float32)),
        mesh=plsc.VectorSubcoreMesh(
            core_axis_name="c", subcore_axis_name="s",
            num_cores=2, num_subcores=16),
        scratch_types=_SCRATCH,
        compiler_params=pltpu.CompilerParams(needs_layout_passes=False),
        interpret=interpret,
    )


def kernel(pred_logits, pred_boxes, tgt_boxes, tgt_labels, src_idx, tgt_idx):
    x = jnp.squeeze(pred_logits, axis=-1)
    _, out = _make_kernel()(
        x.astype(jnp.float32),
        pred_boxes.astype(jnp.float32),
        tgt_boxes.astype(jnp.float32),
        tgt_labels.astype(jnp.int32),
        src_idx.astype(jnp.int32),
        tgt_idx.astype(jnp.int32),
    )
    return (out[0], out[1])


# speedup vs baseline: 1.3560x; 1.0421x over previous
import functools
import math

import jax
import jax.numpy as jnp
from jax import lax
from jax.experimental import pallas as pl
from jax.experimental.pallas import tpu as pltpu
from jax.experimental.pallas import tpu_sc as plsc

_B, _N, _M = 16, 500, 64
_L = 16
_NCHUNK = _N // _L
_TAIL = _N - _NCHUNK * _L
_TBASE = _N - _L


def _sc_body(x_hbm, pb_hbm, tb_hbm, lab_hbm, src_hbm, tgt_hbm,
             part_hbm, out_hbm,
             xv, zv, pbv, tbv, labv, srcv, tgtv, accv, allv, outv, sem):
    cid = lax.axis_index("c")
    sid = lax.axis_index("s")
    row = sid
    zero = jnp.zeros((_L,), jnp.float32)
    lane = lax.iota(jnp.int32, _L)

    cp_src = pltpu.async_copy(src_hbm.at[row], srcv, sem)
    cp_tgt = pltpu.async_copy(tgt_hbm.at[row], tgtv, sem)

    @pl.when(cid == 0)
    def _():
        cp_x = pltpu.async_copy(x_hbm.at[row], xv, sem)
        cp_lab = pltpu.async_copy(lab_hbm.at[row], labv, sem)
        for k in range(_N // _L + 1):
            zv[pl.ds(k * _L, _L)] = zero
        cp_src.wait()
        cp_tgt.wait()
        cp_x.wait()
        cp_lab.wait()

        for k in range(_M // _L):
            svec = srcv[pl.ds(k * _L, _L)]
            tvec = tgtv[pl.ds(k * _L, _L)]
            lab = plsc.load_gather(labv, [tvec]).astype(jnp.float32)
            plsc.store_scatter(zv, [svec], lab)

        def bce(x, z):
            u = jnp.exp(-jnp.abs(x))
            s = u / (u + 2.0)
            s2 = s * s
            lg = s * (2.0 + s2 * (2.0 / 3.0
                                  + s2 * (2.0 / 5.0 + s2 * (2.0 / 7.0))))
            return jnp.maximum(x, 0.0) - x * z + lg

        acc = zero
        for k in range(_NCHUNK):
            acc = acc + bce(xv[pl.ds(k * _L, _L)], zv[pl.ds(k * _L, _L)])
        tail = bce(xv[pl.ds(_TBASE, _L)], zv[pl.ds(_TBASE, _L)])
        accv[...] = acc + jnp.where(lane >= _L - _TAIL, tail, 0.0)

    @pl.when(cid == 1)
    def _():
        cp_pb = pltpu.async_copy(pb_hbm.at[row], pbv, sem)
        cp_tb = pltpu.async_copy(tb_hbm.at[row], tbv, sem)
        cp_src.wait()
        cp_tgt.wait()
        cp_pb.wait()
        cp_tb.wait()

        acc = zero
        for k in range(_M // _L):
            svec = srcv[pl.ds(k * _L, _L)]
            tvec = tgtv[pl.ds(k * _L, _L)]
            for c in range(3):
                cvec = jnp.full((_L,), c, jnp.int32)
                sp = plsc.load_gather(pbv, [svec, cvec])
                tp = plsc.load_gather(tbv, [tvec, cvec])
                acc = acc + jnp.abs(sp - tp)
        accv[...] = acc

    pltpu.sync_copy(accv, part_hbm.at[cid, sid])
    plsc.subcore_barrier()

    @pl.when(sid == 0)
    def _():
        pltpu.sync_copy(part_hbm.at[cid], allv)
        tot = jnp.zeros((_L,), jnp.float32)
        for s_ in range(_B):
            tot = tot + allv[s_, :]
        scale = jnp.where(cid == 0, 1.0 / (_B * _N), 1.0 / (_B * _M))
        loss = jnp.sum(tot) * scale
        outv[...] = jnp.where(lane == 0, loss, 0.0)
        pltpu.sync_copy(outv, out_hbm.at[pl.ds(cid * _L, _L)])


_SCRATCH = [
    pltpu.VMEM((_N,), jnp.float32),
    pltpu.VMEM((_N + _L, ), jnp.float32),
    pltpu.VMEM((_N, 3), jnp.float32),
    pltpu.VMEM((_M, 3), jnp.float32),
    pltpu.VMEM((_M,), jnp.int32),
    pltpu.VMEM((_M,), jnp.int32),
    pltpu.VMEM((_M,), jnp.int32),
    pltpu.VMEM((_L,), jnp.float32),
    pltpu.VMEM((_B, _L), jnp.float32),
    pltpu.VMEM((_L,), jnp.float32),
    pltpu.SemaphoreType.DMA,
]


def _make_kernel(interpret=False):
    return pl.kernel(
        _sc_body,
        out_type=(jax.ShapeDtypeStruct((2, _B, _L), jnp.float32),
                  jax.ShapeDtypeStruct((2 * _L,), jnp.float32)),
        mesh=plsc.VectorSubcoreMesh(
            core_axis_name="c", subcore_axis_name="s",
            num_cores=2, num_subcores=16),
        scratch_types=_SCRATCH,
        compiler_params=pltpu.CompilerParams(needs_layout_passes=False),
        interpret=interpret,
    )


def kernel(pred_logits, pred_boxes, tgt_boxes, tgt_labels, src_idx, tgt_idx):
    x = jnp.squeeze(pred_logits, axis=-1)
    _, out = _make_kernel()(
        x.astype(jnp.float32),
        pred_boxes.astype(jnp.float32),
        tgt_boxes.astype(jnp.float32),
        tgt_labels.astype(jnp.int32),
        src_idx.astype(jnp.int32),
        tgt_idx.astype(jnp.int32),
    )
    return (out[0], out[_L])


# speedup vs baseline: 1.6870x; 1.2442x over previous
import functools
import math

import jax
import jax.numpy as jnp
from jax import lax
from jax.experimental import pallas as pl
from jax.experimental.pallas import tpu as pltpu
from jax.experimental.pallas import tpu_sc as plsc

_B, _N, _M = 16, 500, 64
_L = 16
_NCHUNK = _N // _L
_TAIL = _N - _NCHUNK * _L
_TBASE = _N - _L


def _sc_body(x_hbm, pb_hbm, tb_hbm, lab_hbm, src_hbm, tgt_hbm,
             part_hbm, out_hbm,
             xv, zv, pbv, tbv, labv, srcv, tgtv, accv, allv, outv, sem):
    cid = lax.axis_index("c")
    sid = lax.axis_index("s")
    row = sid
    zero = jnp.zeros((_L,), jnp.float32)
    lane = lax.iota(jnp.int32, _L)

    cp_src = pltpu.async_copy(src_hbm.at[row], srcv, sem)
    cp_tgt = pltpu.async_copy(tgt_hbm.at[row], tgtv, sem)

    @pl.when(cid == 0)
    def _():
        cp_x = pltpu.async_copy(x_hbm.at[row], xv, sem)
        cp_lab = pltpu.async_copy(lab_hbm.at[row], labv, sem)
        for k in range(_N // _L + 1):
            zv[pl.ds(k * _L, _L)] = zero
        cp_src.wait()
        cp_tgt.wait()
        cp_x.wait()
        cp_lab.wait()

        for k in range(_M // _L):
            svec = srcv[pl.ds(k * _L, _L)]
            tvec = tgtv[pl.ds(k * _L, _L)]
            lab = plsc.load_gather(labv, [tvec]).astype(jnp.float32)
            plsc.store_scatter(zv, [svec], lab)

        def bce(x, z):
            u = jnp.exp(-jnp.abs(x))
            s = u / (u + 2.0)
            s2 = s * s
            lg = s * (2.0 + s2 * (2.0 / 3.0
                                  + s2 * (2.0 / 5.0 + s2 * (2.0 / 7.0))))
            return jnp.maximum(x, 0.0) - x * z + lg

        acc = zero
        for k in range(_NCHUNK):
            acc = acc + bce(xv[pl.ds(k * _L, _L)], zv[pl.ds(k * _L, _L)])
        tail = bce(xv[pl.ds(_TBASE, _L)], zv[pl.ds(_TBASE, _L)])
        accv[...] = acc + jnp.where(lane >= _L - _TAIL, tail, 0.0)

    @pl.when(cid == 1)
    def _():
        cps = [pltpu.async_copy(pb_hbm.at[c, pl.ds(row, 1)], pbv.at[c], sem)
               for c in range(3)]
        cps += [pltpu.async_copy(tb_hbm.at[c, pl.ds(row, 1)], tbv.at[c], sem)
                for c in range(3)]
        cp_src.wait()
        cp_tgt.wait()
        for cp in cps:
            cp.wait()

        acc = zero
        for k in range(_M // _L):
            svec = srcv[pl.ds(k * _L, _L)]
            tvec = tgtv[pl.ds(k * _L, _L)]
            zvec = jnp.zeros((_L,), jnp.int32)
            for c in range(3):
                cvec = jnp.full((_L,), c, jnp.int32)
                sp = plsc.load_gather(pbv, [cvec, zvec, svec])
                tp = plsc.load_gather(tbv, [cvec, zvec, tvec])
                acc = acc + jnp.abs(sp - tp)
        accv[...] = acc

    pltpu.sync_copy(accv, part_hbm.at[cid, sid])
    plsc.subcore_barrier()

    @pl.when(sid == 0)
    def _():
        pltpu.sync_copy(part_hbm.at[cid], allv)
        tot = jnp.zeros((_L,), jnp.float32)
        for s_ in range(_B):
            tot = tot + allv[s_, :]
        scale = jnp.where(cid == 0, 1.0 / (_B * _N), 1.0 / (_B * _M))
        loss = jnp.sum(tot) * scale
        outv[...] = jnp.where(lane == 0, loss, 0.0)
        pltpu.sync_copy(outv, out_hbm.at[pl.ds(cid * _L, _L)])


_SCRATCH = [
    pltpu.VMEM((_N,), jnp.float32),
    pltpu.VMEM((_N + _L, ), jnp.float32),
    pltpu.VMEM((3, 1, _N), jnp.float32),
    pltpu.VMEM((3, 1, _M), jnp.float32),
    pltpu.VMEM((_M,), jnp.int32),
    pltpu.VMEM((_M,), jnp.int32),
    pltpu.VMEM((_M,), jnp.int32),
    pltpu.VMEM((_L,), jnp.float32),
    pltpu.VMEM((_B, _L), jnp.float32),
    pltpu.VMEM((_L,), jnp.float32),
    pltpu.SemaphoreType.DMA,
]


def _make_kernel(interpret=False):
    return pl.kernel(
        _sc_body,
        out_type=(jax.ShapeDtypeStruct((2, _B, _L), jnp.float32),
                  jax.ShapeDtypeStruct((2 * _L,), jnp.float32)),
        mesh=plsc.VectorSubcoreMesh(
            core_axis_name="c", subcore_axis_name="s",
            num_cores=2, num_subcores=16),
        scratch_types=_SCRATCH,
        compiler_params=pltpu.CompilerParams(needs_layout_passes=False),
        interpret=interpret,
    )


def kernel(pred_logits, pred_boxes, tgt_boxes, tgt_labels, src_idx, tgt_idx):
    x = jnp.squeeze(pred_logits, axis=-1)
    _, out = _make_kernel()(
        x.astype(jnp.float32),
        jnp.transpose(pred_boxes.astype(jnp.float32), (2, 0, 1)),
        jnp.transpose(tgt_boxes.astype(jnp.float32), (2, 0, 1)),
        tgt_labels.astype(jnp.int32),
        src_idx.astype(jnp.int32),
        tgt_idx.astype(jnp.int32),
    )
    return (out[0], out[_L])
